# baseline (speedup 1.0000x reference)
import functools

import jax
import jax.numpy as jnp
from jax import lax
from jax.experimental import pallas as pl
from jax.experimental.pallas import tpu as pltpu
from jax.experimental.pallas import tpu_sc as plsc

N = 10000
D = 128
E = 320000
NC, NS = 2, 16
NPAD = 10240
NACC = 10112
RPA = NACC // NS
EPAD = 327680
EPT = EPAD // (NC * NS)
EPC = EPT * NS
CHUNK = 64
CPT = EPAD // NC // NS // CHUNK
KBUF = 4
KBI = 5
BLK = 1024

_MESH = plsc.VectorSubcoreMesh(
    core_axis_name="c", subcore_axis_name="s", num_cores=NC, num_subcores=NS
)


def _deg_body(src_hbm, dst_hbm, zflat_hbm, osrc_hbm, odst_hbm,
              sidx, didx, hs_v, hd_v):
    c = lax.axis_index("c")
    s = lax.axis_index("s")
    pltpu.sync_copy(zflat_hbm, hs_v)
    pltpu.sync_copy(zflat_hbm, hd_v)
    ebase = c * EPC + s * EPT
    pltpu.sync_copy(src_hbm.at[pl.ds(ebase, EPT)], sidx)
    pltpu.sync_copy(dst_hbm.at[pl.ds(ebase, EPT)], didx)
    ones = jnp.ones((16,), jnp.float32)

    @pl.loop(0, EPT // 16, unroll=4)
    def _body(j):
        plsc.addupdate_scatter(hs_v, [sidx[pl.ds(j * 16, 16)]], ones)
        plsc.addupdate_scatter(hd_v, [didx[pl.ds(j * 16, 16)]], ones)

    w = c * NS + s
    pltpu.sync_copy(hs_v, osrc_hbm.at[w])
    pltpu.sync_copy(hd_v, odst_hbm.at[w])


_DEG_OUT = (
    jax.ShapeDtypeStruct((NC * NS, NPAD), jnp.float32),
    jax.ShapeDtypeStruct((NC * NS, NPAD), jnp.float32),
)
_DEG_SCRATCH = [
    pltpu.VMEM((EPT,), jnp.int32),
    pltpu.VMEM((EPT,), jnp.int32),
    pltpu.VMEM((NPAD,), jnp.float32),
    pltpu.VMEM((NPAD,), jnp.float32),
]
_deg_kernel = pl.kernel(
    _deg_body, out_type=_DEG_OUT, mesh=_MESH, scratch_types=_DEG_SCRATCH,
    compiler_params=pltpu.CompilerParams(needs_layout_passes=False),
)


def _agg_body(y_hbm, e2_hbm, zrow_hbm, out_hbm,
              idxbuf, rows, acc_sh,
              g0, g1, g2, g3, s0, s1, s2, s3, i0, i1, i2, i3, i4):
    c = lax.axis_index("c")
    s = lax.axis_index("s")
    gsem = (g0, g1, g2, g3)
    ssem = (s0, s1, s2, s3)
    isem = (i0, i1, i2, i3, i4)
    r0 = s * RPA
    pltpu.sync_copy(zrow_hbm, acc_sh.at[pl.ds(r0, RPA)])
    plsc.subcore_barrier()
    cbase = c * (NS * CPT) + s * CPT

    def idxload(cc, i):
        pltpu.async_copy(e2_hbm.at[cbase + cc], idxbuf.at[i], isem[i])

    def wait_idx(i):
        pltpu.make_async_copy(e2_hbm.at[cbase], idxbuf.at[i],
                              isem[i]).wait()

    def gather(cc_unused, b, i):
        pltpu.async_copy(y_hbm.at[idxbuf.at[i, 0]], rows.at[b], gsem[b])

    def wait_gather(b):
        pltpu.make_async_copy(y_hbm.at[idxbuf.at[0, 0]], rows.at[b],
                              gsem[b]).wait()

    def scatter(b, i):
        pltpu.async_copy(rows.at[b], acc_sh.at[idxbuf.at[i, 1]],
                         ssem[b], add=True)

    def wait_scatter(b):
        pltpu.make_async_copy(rows.at[b], acc_sh.at[idxbuf.at[0, 1]],
                              ssem[b]).wait()

    def handler(cc, k, w_ssem=True, do_idx=True, do_scatter=True):
        b, i = k % KBUF, k % KBI
        if w_ssem:
            wait_scatter(b)
        if do_idx:
            idxload(cc + 1, (i + 1) % KBI)
        wait_idx(i)
        gather(cc, b, i)
        if do_scatter:
            wait_gather((k - 2) % KBUF)
            scatter((k - 2) % KBUF, (k - 2) % KBI)

    idxload(0, 0)
    handler(0, 0, w_ssem=False, do_scatter=False)
    handler(1, 1, w_ssem=False, do_scatter=False)
    handler(2, 2, w_ssem=False)
    handler(3, 3, w_ssem=False)

    @pl.loop(4, 144, step=20)
    def _steady(j):
        for k in range(20):
            handler(j + k, 4 + k)

    for cc in range(144, 159):
        handler(cc, cc)
    handler(159, 159, do_idx=False)
    wait_gather(158 % KBUF)
    scatter(158 % KBUF, 158 % KBI)
    wait_gather(159 % KBUF)
    scatter(159 % KBUF, 159 % KBI)
    for b in range(KBUF):
        wait_scatter(b)

    plsc.subcore_barrier()
    pltpu.sync_copy(acc_sh.at[pl.ds(r0, RPA)], out_hbm.at[c, pl.ds(r0, RPA)])


_AGG_OUT = jax.ShapeDtypeStruct((NC, NPAD, D), jnp.float32)
_AGG_SCRATCH = [
    pltpu.VMEM((KBI, 2, CHUNK), jnp.int32),
    pltpu.VMEM((KBUF, CHUNK, D), jnp.float32),
    pltpu.VMEM_SHARED((NACC, D), jnp.float32),
] + [pltpu.SemaphoreType.DMA] * (KBUF + KBUF + KBI)
_agg_kernel = pl.kernel(
    _agg_body, out_type=_AGG_OUT, mesh=_MESH, scratch_types=_AGG_SCRATCH
)


def _prescale_body(x_ref, hs_ref, hd_ref, y_ref, ns_ref, nd_ref):
    degs = jnp.sum(jnp.transpose(hs_ref[...]), axis=1, keepdims=True)
    degd = jnp.sum(jnp.transpose(hd_ref[...]), axis=1, keepdims=True)
    ns = lax.rsqrt(jnp.maximum(degs, 1.0))
    nd = lax.rsqrt(jnp.maximum(degd, 1.0))
    ns_ref[...] = jnp.broadcast_to(ns, (BLK, 16))
    nd_ref[...] = jnp.broadcast_to(nd, (BLK, 16))
    y_ref[...] = x_ref[...] * ns


def _prescale(xp, hs, hd):
    grid = NPAD // BLK
    return pl.pallas_call(
        _prescale_body,
        grid=(grid,),
        in_specs=[
            pl.BlockSpec((BLK, D), lambda i: (i, 0)),
            pl.BlockSpec((NC * NS, BLK), lambda i: (0, i)),
            pl.BlockSpec((NC * NS, BLK), lambda i: (0, i)),
        ],
        out_specs=[
            pl.BlockSpec((BLK, D), lambda i: (i, 0)),
            pl.BlockSpec((BLK, 16), lambda i: (i, 0)),
            pl.BlockSpec((BLK, 16), lambda i: (i, 0)),
        ],
        out_shape=[
            jax.ShapeDtypeStruct((NPAD, D), jnp.float32),
            jax.ShapeDtypeStruct((NPAD, 16), jnp.float32),
            jax.ShapeDtypeStruct((NPAD, 16), jnp.float32),
        ],
    )(xp, hs, hd)


def _dense_body(acc_ref, nd_ref, ns_ref, w_ref, o_ref, *, last):
    a = acc_ref[0] + acc_ref[1]
    h = jnp.dot(a * nd_ref[:, :1], w_ref[...],
                preferred_element_type=jnp.float32)
    if not last:
        h = jnp.maximum(h, 0.0) * ns_ref[:, :1]
    i = pl.program_id(0)
    row = i * BLK + lax.broadcasted_iota(jnp.int32, (BLK, 1), 0)
    o_ref[...] = jnp.where(row < NACC, h, 0.0)


def _dense(acc, nd16, ns16, W, last):
    grid = NPAD // BLK
    return pl.pallas_call(
        functools.partial(_dense_body, last=last),
        grid=(grid,),
        in_specs=[
            pl.BlockSpec((NC, BLK, D), lambda i: (0, i, 0)),
            pl.BlockSpec((BLK, 16), lambda i: (i, 0)),
            pl.BlockSpec((BLK, 16), lambda i: (i, 0)),
            pl.BlockSpec((D, D), lambda i: (0, 0)),
        ],
        out_specs=pl.BlockSpec((BLK, D), lambda i: (i, 0)),
        out_shape=jax.ShapeDtypeStruct((NPAD, D), jnp.float32),
    )(acc, nd16, ns16, W)


def kernel(features, edge_index, W1, W2, W3):
    xp = jnp.zeros((NPAD, D), jnp.float32).at[:N].set(features)
    pad_i = jnp.arange(EPAD - E, dtype=jnp.int32)
    pad_dummy = N + pad_i % (NPAD - N)
    src = jnp.concatenate([edge_index[0], pad_dummy])
    dst = jnp.concatenate([edge_index[1], pad_dummy])
    src_a = jnp.concatenate([edge_index[0], pad_dummy])
    dst_a = jnp.concatenate([edge_index[1], pad_i % N])
    e2 = jnp.stack(
        [src_a.reshape(EPAD // CHUNK, CHUNK),
         dst_a.reshape(EPAD // CHUNK, CHUNK)],
        axis=1)
    zflat = jnp.zeros((NPAD,), jnp.float32)
    zrow = jnp.zeros((RPA, D), jnp.float32)

    hs, hd = _deg_kernel(src, dst, zflat)
    y, ns16, nd16 = _prescale(xp, hs, hd)
    for W, last in ((W1, False), (W2, False), (W3, True)):
        acc = _agg_kernel(y, e2, zrow)
        y = _dense(acc, nd16, ns16, W, last)
    return y[:N]


# speedup vs baseline: 1.0692x; 1.0692x over previous
import functools

import jax
import jax.numpy as jnp
from jax import lax
from jax.experimental import pallas as pl
from jax.experimental.pallas import tpu as pltpu
from jax.experimental.pallas import tpu_sc as plsc

N = 10000
D = 128
E = 320000
NC, NS = 2, 16
NPAD = 10240
NACC = 10016
RPA = 632
RPA_L = NACC - (NS - 1) * RPA
EPAD = 327680
EPT = EPAD // (NC * NS)
EPC = EPT * NS
CHUNK = 128
CPT = EPAD // NC // NS // CHUNK
KBUF = 3
KBI = 4
BLK = 1024

_MESH = plsc.VectorSubcoreMesh(
    core_axis_name="c", subcore_axis_name="s", num_cores=NC, num_subcores=NS
)


def _deg_body(src_hbm, dst_hbm, zflat_hbm, osrc_hbm, odst_hbm,
              sidx, didx, hs_v, hd_v):
    c = lax.axis_index("c")
    s = lax.axis_index("s")
    pltpu.sync_copy(zflat_hbm, hs_v)
    pltpu.sync_copy(zflat_hbm, hd_v)
    ebase = c * EPC + s * EPT
    pltpu.sync_copy(src_hbm.at[pl.ds(ebase, EPT)], sidx)
    pltpu.sync_copy(dst_hbm.at[pl.ds(ebase, EPT)], didx)
    ones = jnp.ones((16,), jnp.float32)

    @pl.loop(0, EPT // 16, unroll=8)
    def _body(j):
        plsc.addupdate_scatter(hs_v, [sidx[pl.ds(j * 16, 16)]], ones)
        plsc.addupdate_scatter(hd_v, [didx[pl.ds(j * 16, 16)]], ones)

    w = c * NS + s
    pltpu.sync_copy(hs_v, osrc_hbm.at[w])
    pltpu.sync_copy(hd_v, odst_hbm.at[w])


_DEG_OUT = (
    jax.ShapeDtypeStruct((NC * NS, NPAD), jnp.float32),
    jax.ShapeDtypeStruct((NC * NS, NPAD), jnp.float32),
)
_DEG_SCRATCH = [
    pltpu.VMEM((EPT,), jnp.int32),
    pltpu.VMEM((EPT,), jnp.int32),
    pltpu.VMEM((NPAD,), jnp.float32),
    pltpu.VMEM((NPAD,), jnp.float32),
]
_deg_kernel = pl.kernel(
    _deg_body, out_type=_DEG_OUT, mesh=_MESH, scratch_types=_DEG_SCRATCH,
    compiler_params=pltpu.CompilerParams(needs_layout_passes=False),
)


def _agg_body(y_hbm, e2_hbm, zrow_hbm, out_hbm,
              idxbuf, rows, acc_sh,
              g0, g1, g2, s0, s1, s2, i0, i1, i2, i3):
    c = lax.axis_index("c")
    s = lax.axis_index("s")
    gsem = (g0, g1, g2)
    ssem = (s0, s1, s2)
    isem = (i0, i1, i2, i3)
    r0 = s * RPA

    @pl.when(s < NS - 1)
    def _zf():
        pltpu.sync_copy(zrow_hbm, acc_sh.at[pl.ds(r0, RPA)])

    @pl.when(s == NS - 1)
    def _zl():
        pltpu.sync_copy(zrow_hbm.at[pl.ds(0, RPA_L)],
                        acc_sh.at[pl.ds((NS - 1) * RPA, RPA_L)])

    plsc.subcore_barrier()
    cbase = c * (NS * CPT) + s * CPT

    def idxload(cc, i):
        pltpu.async_copy(e2_hbm.at[cbase + cc], idxbuf.at[i], isem[i])

    def wait_idx(i):
        pltpu.make_async_copy(e2_hbm.at[cbase], idxbuf.at[i],
                              isem[i]).wait()

    def gather(cc_unused, b, i):
        pltpu.async_copy(y_hbm.at[idxbuf.at[i, 0]], rows.at[b], gsem[b])

    def wait_gather(b):
        pltpu.make_async_copy(y_hbm.at[idxbuf.at[0, 0]], rows.at[b],
                              gsem[b]).wait()

    def scatter(b, i):
        pltpu.async_copy(rows.at[b], acc_sh.at[idxbuf.at[i, 1]],
                         ssem[b], add=True)

    def wait_scatter(b):
        pltpu.make_async_copy(rows.at[b], acc_sh.at[idxbuf.at[0, 1]],
                              ssem[b]).wait()

    def handler(cc, k, w_ssem=True, do_idx=True, do_scatter=True):
        b, i = k % KBUF, k % KBI
        if w_ssem:
            wait_scatter(b)
        if do_idx:
            idxload(cc + 1, (i + 1) % KBI)
        wait_idx(i)
        gather(cc, b, i)
        if do_scatter:
            wait_gather((k - 2) % KBUF)
            scatter((k - 2) % KBUF, (k - 2) % KBI)

    idxload(0, 0)
    handler(0, 0, w_ssem=False, do_scatter=False)
    handler(1, 1, w_ssem=False, do_scatter=False)
    handler(2, 2, w_ssem=False)

    @pl.loop(3, 75, step=12)
    def _steady(j):
        for k in range(12):
            handler(j + k, 3 + k)

    for cc in range(75, 79):
        handler(cc, cc)
    handler(79, 79, do_idx=False)
    wait_gather(78 % KBUF)
    scatter(78 % KBUF, 78 % KBI)
    wait_gather(79 % KBUF)
    scatter(79 % KBUF, 79 % KBI)
    for b in (77 % KBUF, 78 % KBUF, 79 % KBUF):
        wait_scatter(b)

    plsc.subcore_barrier()

    @pl.when(s < NS - 1)
    def _wf():
        pltpu.sync_copy(acc_sh.at[pl.ds(r0, RPA)],
                        out_hbm.at[c, pl.ds(r0, RPA)])

    @pl.when(s == NS - 1)
    def _wl():
        pltpu.sync_copy(acc_sh.at[pl.ds((NS - 1) * RPA, RPA_L)],
                        out_hbm.at[c, pl.ds((NS - 1) * RPA, RPA_L)])


_AGG_OUT = jax.ShapeDtypeStruct((NC, NPAD, D), jnp.float32)
_AGG_SCRATCH = [
    pltpu.VMEM((KBI, 2, CHUNK), jnp.int32),
    pltpu.VMEM((KBUF, CHUNK, D), jnp.float32),
    pltpu.VMEM_SHARED((NACC, D), jnp.float32),
    pltpu.SemaphoreType.DMA,
    pltpu.SemaphoreType.DMA,
    pltpu.SemaphoreType.DMA,
    pltpu.SemaphoreType.DMA,
    pltpu.SemaphoreType.DMA,
    pltpu.SemaphoreType.DMA,
    pltpu.SemaphoreType.DMA,
    pltpu.SemaphoreType.DMA,
    pltpu.SemaphoreType.DMA,
    pltpu.SemaphoreType.DMA,
]
_agg_kernel = pl.kernel(
    _agg_body, out_type=_AGG_OUT, mesh=_MESH, scratch_types=_AGG_SCRATCH
)


def _prescale_body(x_ref, hs_ref, hd_ref, y_ref, ns_ref, nd_ref):
    degs = jnp.sum(jnp.transpose(hs_ref[...]), axis=1, keepdims=True)
    degd = jnp.sum(jnp.transpose(hd_ref[...]), axis=1, keepdims=True)
    ns = lax.rsqrt(jnp.maximum(degs, 1.0))
    nd = lax.rsqrt(jnp.maximum(degd, 1.0))
    ns_ref[...] = jnp.broadcast_to(ns, (BLK, 16))
    nd_ref[...] = jnp.broadcast_to(nd, (BLK, 16))
    y_ref[...] = x_ref[...] * ns


def _prescale(xp, hs, hd):
    grid = NPAD // BLK
    return pl.pallas_call(
        _prescale_body,
        grid=(grid,),
        in_specs=[
            pl.BlockSpec((BLK, D), lambda i: (i, 0)),
            pl.BlockSpec((NC * NS, BLK), lambda i: (0, i)),
            pl.BlockSpec((NC * NS, BLK), lambda i: (0, i)),
        ],
        out_specs=[
            pl.BlockSpec((BLK, D), lambda i: (i, 0)),
            pl.BlockSpec((BLK, 16), lambda i: (i, 0)),
            pl.BlockSpec((BLK, 16), lambda i: (i, 0)),
        ],
        out_shape=[
            jax.ShapeDtypeStruct((NPAD, D), jnp.float32),
            jax.ShapeDtypeStruct((NPAD, 16), jnp.float32),
            jax.ShapeDtypeStruct((NPAD, 16), jnp.float32),
        ],
    )(xp, hs, hd)


def _dense_body(acc_ref, nd_ref, ns_ref, w_ref, o_ref, *, last):
    a = acc_ref[0] + acc_ref[1]
    h = jnp.dot(a * nd_ref[:, :1], w_ref[...],
                preferred_element_type=jnp.float32)
    if not last:
        h = jnp.maximum(h, 0.0) * ns_ref[:, :1]
    i = pl.program_id(0)
    row = i * BLK + lax.broadcasted_iota(jnp.int32, (BLK, 1), 0)
    o_ref[...] = jnp.where(row < NACC, h, 0.0)


def _dense(acc, nd16, ns16, W, last):
    grid = NPAD // BLK
    return pl.pallas_call(
        functools.partial(_dense_body, last=last),
        grid=(grid,),
        in_specs=[
            pl.BlockSpec((NC, BLK, D), lambda i: (0, i, 0)),
            pl.BlockSpec((BLK, 16), lambda i: (i, 0)),
            pl.BlockSpec((BLK, 16), lambda i: (i, 0)),
            pl.BlockSpec((D, D), lambda i: (0, 0)),
        ],
        out_specs=pl.BlockSpec((BLK, D), lambda i: (i, 0)),
        out_shape=jax.ShapeDtypeStruct((NPAD, D), jnp.float32),
    )(acc, nd16, ns16, W)


def kernel(features, edge_index, W1, W2, W3):
    xp = jnp.zeros((NPAD, D), jnp.float32).at[:N].set(features)
    pad_i = jnp.arange(EPAD - E, dtype=jnp.int32)
    pad_dummy = N + pad_i % (NPAD - N)
    src = jnp.concatenate([edge_index[0], pad_dummy])
    dst = jnp.concatenate([edge_index[1], pad_dummy])
    src_a = jnp.concatenate([edge_index[0], pad_dummy])
    dst_a = jnp.concatenate([edge_index[1], pad_i % N])
    e2 = jnp.stack(
        [src_a.reshape(EPAD // CHUNK, CHUNK),
         dst_a.reshape(EPAD // CHUNK, CHUNK)],
        axis=1)
    zflat = jnp.zeros((NPAD,), jnp.float32)
    zrow = jnp.zeros((RPA, D), jnp.float32)

    hs, hd = _deg_kernel(src, dst, zflat)
    y, ns16, nd16 = _prescale(xp, hs, hd)
    for W, last in ((W1, False), (W2, False), (W3, True)):
        acc = _agg_kernel(y, e2, zrow)
        y = _dense(acc, nd16, ns16, W, last)
    return y[:N]


# speedup vs baseline: 1.0934x; 1.0227x over previous
import functools

import jax
import jax.numpy as jnp
from jax import lax
from jax.experimental import pallas as pl
from jax.experimental.pallas import tpu as pltpu
from jax.experimental.pallas import tpu_sc as plsc

N = 10000
D = 128
E = 320000
NC, NS = 2, 16
NPAD = 10240
NACC = 10016
RPA = 632
RPA_L = NACC - (NS - 1) * RPA
EPAD = 327680
EPT = EPAD // (NC * NS)
EPC = EPT * NS
CHUNK = 128
CPT = EPAD // NC // NS // CHUNK
KBUF = 3
KBI = 4
BLK = 1024

_MESH = plsc.VectorSubcoreMesh(
    core_axis_name="c", subcore_axis_name="s", num_cores=NC, num_subcores=NS
)


def _deg_body(src_hbm, dst_hbm, zflat_hbm, osrc_hbm, odst_hbm,
              sidx, didx, hs_v, hd_v):
    c = lax.axis_index("c")
    s = lax.axis_index("s")
    pltpu.sync_copy(zflat_hbm, hs_v)
    pltpu.sync_copy(zflat_hbm, hd_v)
    ebase = c * EPC + s * EPT
    pltpu.sync_copy(src_hbm.at[pl.ds(ebase, EPT)], sidx)
    pltpu.sync_copy(dst_hbm.at[pl.ds(ebase, EPT)], didx)
    ones = jnp.ones((16,), jnp.float32)

    @pl.loop(0, EPT // 16, unroll=8)
    def _body(j):
        plsc.addupdate_scatter(hs_v, [sidx[pl.ds(j * 16, 16)]], ones)
        plsc.addupdate_scatter(hd_v, [didx[pl.ds(j * 16, 16)]], ones)

    w = c * NS + s
    pltpu.sync_copy(hs_v, osrc_hbm.at[w])
    pltpu.sync_copy(hd_v, odst_hbm.at[w])


_DEG_OUT = (
    jax.ShapeDtypeStruct((NC * NS, NPAD), jnp.float32),
    jax.ShapeDtypeStruct((NC * NS, NPAD), jnp.float32),
)
_DEG_SCRATCH = [
    pltpu.VMEM((EPT,), jnp.int32),
    pltpu.VMEM((EPT,), jnp.int32),
    pltpu.VMEM((NPAD,), jnp.float32),
    pltpu.VMEM((NPAD,), jnp.float32),
]
_deg_kernel = pl.kernel(
    _deg_body, out_type=_DEG_OUT, mesh=_MESH, scratch_types=_DEG_SCRATCH,
    compiler_params=pltpu.CompilerParams(needs_layout_passes=False),
)


def _agg_body(y_hbm, e2_hbm, zrow_hbm, out_hbm,
              idxbuf, rows, acc_sh,
              g0, g1, g2, s0, s1, s2, i0, i1, i2, i3):
    c = lax.axis_index("c")
    s = lax.axis_index("s")
    gsem = (g0, g1, g2)
    ssem = (s0, s1, s2)
    isem = (i0, i1, i2, i3)
    r0 = s * RPA
    pltpu.sync_copy(zrow_hbm, rows.at[0])
    for k in range(RPA // CHUNK):
        pltpu.sync_copy(rows.at[0], acc_sh.at[pl.ds(r0 + k * CHUNK, CHUNK)])

    @pl.when(s < NS - 1)
    def _zf():
        pltpu.sync_copy(rows.at[0, pl.ds(0, RPA % CHUNK)],
                        acc_sh.at[pl.ds(r0 + RPA - RPA % CHUNK, RPA % CHUNK)])

    @pl.when(s == NS - 1)
    def _zl():
        pltpu.sync_copy(rows.at[0, pl.ds(0, RPA_L % CHUNK)],
                        acc_sh.at[pl.ds((NS - 1) * RPA + RPA_L - RPA_L % CHUNK,
                                        RPA_L % CHUNK)])

    plsc.subcore_barrier()
    cbase = c * (NS * CPT) + s * CPT

    def idxload(cc, i):
        pltpu.async_copy(e2_hbm.at[cbase + cc], idxbuf.at[i], isem[i])

    def wait_idx(i):
        pltpu.make_async_copy(e2_hbm.at[cbase], idxbuf.at[i],
                              isem[i]).wait()

    def gather(cc_unused, b, i):
        pltpu.async_copy(y_hbm.at[idxbuf.at[i, 0]], rows.at[b], gsem[b])

    def wait_gather(b):
        pltpu.make_async_copy(y_hbm.at[idxbuf.at[0, 0]], rows.at[b],
                              gsem[b]).wait()

    def scatter(b, i):
        pltpu.async_copy(rows.at[b], acc_sh.at[idxbuf.at[i, 1]],
                         ssem[b], add=True)

    def wait_scatter(b):
        pltpu.make_async_copy(rows.at[b], acc_sh.at[idxbuf.at[0, 1]],
                              ssem[b]).wait()

    def handler(cc, k, w_ssem=True, do_idx=True, do_scatter=True):
        b, i = k % KBUF, k % KBI
        if w_ssem:
            wait_scatter(b)
        if do_idx:
            idxload(cc + 1, (i + 1) % KBI)
        wait_idx(i)
        gather(cc, b, i)
        if do_scatter:
            wait_gather((k - 2) % KBUF)
            scatter((k - 2) % KBUF, (k - 2) % KBI)

    idxload(0, 0)
    handler(0, 0, w_ssem=False, do_scatter=False)
    handler(1, 1, w_ssem=False, do_scatter=False)
    handler(2, 2, w_ssem=False)

    @pl.loop(3, 75, step=12)
    def _steady(j):
        for k in range(12):
            handler(j + k, 3 + k)

    for cc in range(75, 79):
        handler(cc, cc)
    handler(79, 79, do_idx=False)
    wait_gather(78 % KBUF)
    scatter(78 % KBUF, 78 % KBI)
    wait_gather(79 % KBUF)
    scatter(79 % KBUF, 79 % KBI)
    for b in (77 % KBUF, 78 % KBUF, 79 % KBUF):
        wait_scatter(b)

    plsc.subcore_barrier()

    @pl.when(s < NS - 1)
    def _wf():
        pltpu.sync_copy(acc_sh.at[pl.ds(r0, RPA)],
                        out_hbm.at[c, pl.ds(r0, RPA)])

    @pl.when(s == NS - 1)
    def _wl():
        pltpu.sync_copy(acc_sh.at[pl.ds((NS - 1) * RPA, RPA_L)],
                        out_hbm.at[c, pl.ds((NS - 1) * RPA, RPA_L)])


_AGG_OUT = jax.ShapeDtypeStruct((NC, NPAD, D), jnp.float32)
_AGG_SCRATCH = [
    pltpu.VMEM((KBI, 2, CHUNK), jnp.int32),
    pltpu.VMEM((KBUF, CHUNK, D), jnp.float32),
    pltpu.VMEM_SHARED((NACC, D), jnp.float32),
    pltpu.SemaphoreType.DMA,
    pltpu.SemaphoreType.DMA,
    pltpu.SemaphoreType.DMA,
    pltpu.SemaphoreType.DMA,
    pltpu.SemaphoreType.DMA,
    pltpu.SemaphoreType.DMA,
    pltpu.SemaphoreType.DMA,
    pltpu.SemaphoreType.DMA,
    pltpu.SemaphoreType.DMA,
    pltpu.SemaphoreType.DMA,
]
_agg_kernel = pl.kernel(
    _agg_body, out_type=_AGG_OUT, mesh=_MESH, scratch_types=_AGG_SCRATCH
)


def _prescale_body(x_ref, hs_ref, hd_ref, y_ref, ns_ref, nd_ref):
    degs = jnp.sum(jnp.transpose(hs_ref[...]), axis=1, keepdims=True)
    degd = jnp.sum(jnp.transpose(hd_ref[...]), axis=1, keepdims=True)
    ns = lax.rsqrt(jnp.maximum(degs, 1.0))
    nd = lax.rsqrt(jnp.maximum(degd, 1.0))
    ns_ref[...] = jnp.broadcast_to(ns, (BLK, 16))
    nd_ref[...] = jnp.broadcast_to(nd, (BLK, 16))
    y_ref[...] = x_ref[...] * ns


def _prescale(xp, hs, hd):
    grid = NPAD // BLK
    return pl.pallas_call(
        _prescale_body,
        grid=(grid,),
        in_specs=[
            pl.BlockSpec((BLK, D), lambda i: (i, 0)),
            pl.BlockSpec((NC * NS, BLK), lambda i: (0, i)),
            pl.BlockSpec((NC * NS, BLK), lambda i: (0, i)),
        ],
        out_specs=[
            pl.BlockSpec((BLK, D), lambda i: (i, 0)),
            pl.BlockSpec((BLK, 16), lambda i: (i, 0)),
            pl.BlockSpec((BLK, 16), lambda i: (i, 0)),
        ],
        out_shape=[
            jax.ShapeDtypeStruct((NPAD, D), jnp.float32),
            jax.ShapeDtypeStruct((NPAD, 16), jnp.float32),
            jax.ShapeDtypeStruct((NPAD, 16), jnp.float32),
        ],
    )(xp, hs, hd)


def _dense_body(acc_ref, nd_ref, ns_ref, w_ref, o_ref, *, last, blk):
    a = acc_ref[0] + acc_ref[1]
    h = jnp.dot(a * nd_ref[:, :1], w_ref[...],
                preferred_element_type=jnp.float32)
    if last:
        o_ref[...] = h
    else:
        h = jnp.maximum(h, 0.0) * ns_ref[:, :1]
        i = pl.program_id(0)
        row = i * blk + lax.broadcasted_iota(jnp.int32, (blk, 1), 0)
        o_ref[...] = jnp.where(row < NACC, h, 0.0)


def _dense(acc, nd16, ns16, W, last):
    blk, rows_out = (1000, N) if last else (BLK, NPAD)
    return pl.pallas_call(
        functools.partial(_dense_body, last=last, blk=blk),
        grid=(rows_out // blk,),
        in_specs=[
            pl.BlockSpec((NC, blk, D), lambda i: (0, i, 0)),
            pl.BlockSpec((blk, 16), lambda i: (i, 0)),
            pl.BlockSpec((blk, 16), lambda i: (i, 0)),
            pl.BlockSpec((D, D), lambda i: (0, 0)),
        ],
        out_specs=pl.BlockSpec((blk, D), lambda i: (i, 0)),
        out_shape=jax.ShapeDtypeStruct((rows_out, D), jnp.float32),
    )(acc, nd16, ns16, W)


def kernel(features, edge_index, W1, W2, W3):
    xp = jnp.zeros((NPAD, D), jnp.float32).at[:N].set(features)
    pad_i = jnp.arange(EPAD - E, dtype=jnp.int32)
    pad_dummy = N + pad_i % (NPAD - N)
    src = jnp.concatenate([edge_index[0], pad_dummy])
    dst = jnp.concatenate([edge_index[1], pad_dummy])
    src_a = jnp.concatenate([edge_index[0], pad_dummy])
    dst_a = jnp.concatenate([edge_index[1], pad_i % N])
    e2 = jnp.stack(
        [src_a.reshape(EPAD // CHUNK, CHUNK),
         dst_a.reshape(EPAD // CHUNK, CHUNK)],
        axis=1)
    zflat = jnp.zeros((NPAD,), jnp.float32)
    zrow = jnp.zeros((CHUNK, D), jnp.float32)

    hs, hd = _deg_kernel(src, dst, zflat)
    y, ns16, nd16 = _prescale(xp, hs, hd)
    for W, last in ((W1, False), (W2, False), (W3, True)):
        acc = _agg_kernel(y, e2, zrow)
        y = _dense(acc, nd16, ns16, W, last)
    return y
